# baseline (speedup 1.0000x reference)
import jax
import jax.numpy as jnp
from jax import lax
from jax.experimental import pallas as pl
from jax.experimental.pallas import tpu as pltpu
from jax.experimental.pallas import tpu_sc as plsc

NC = 2
NS = 16
NW = NC * NS
LANES = 16
SR = 8
NQ = 4
QW = 1024


def _permute_body(B, D, u_hbm, perm_hbm, out_hbm,
                  idx_v, in0, in1, in2, ob0, ob1,
                  isem0, isem1, isem2, osem0, osem1):
    rows_per_w = B // NW
    n_stripes = rows_per_w // SR

    wid = lax.axis_index("s") * NC + lax.axis_index("c")
    base = wid * rows_per_w

    in_bufs, isems = (in0, in1, in2), (isem0, isem1, isem2)
    out_bufs, osems = (ob0, ob1), (osem0, osem1)

    pltpu.sync_copy(perm_hbm, idx_v)

    for b in range(3):
        pltpu.async_copy(u_hbm.at[pl.ds(base + b * SR, SR)],
                         in_bufs[b], isems[b])

    def stripe(k, b):
        row0 = base + k * SR
        in_v = in_bufs[b]
        pltpu.make_async_copy(u_hbm.at[pl.ds(row0, SR)],
                              in_v, isems[b]).wait()
        for q in range(NQ):
            ob = q % 2
            out_v = out_bufs[ob]

            @pl.when((k >= 1) | (q >= 2))
            def _():
                pltpu.make_async_copy(
                    out_v, out_hbm.at[pl.ds(row0, SR), pl.ds(q * QW, QW)],
                    osems[ob]).wait()

            @plsc.parallel_loop(0, QW // LANES, unroll=2)
            def _(j):
                j16 = j * LANES
                idx16 = idx_v[pl.ds(q * QW + j16, LANES)]
                for r in range(SR):
                    rvec = jnp.full((LANES,), r, jnp.int32)
                    vals = plsc.load_gather(in_v, [rvec, idx16])
                    out_v[r, pl.ds(j16, LANES)] = vals

            pltpu.async_copy(
                out_v, out_hbm.at[pl.ds(row0, SR), pl.ds(q * QW, QW)],
                osems[ob])

        @pl.when(k + 3 < n_stripes)
        def _():
            pltpu.async_copy(u_hbm.at[pl.ds(row0 + 3 * SR, SR)],
                             in_v, isems[b])

    def trip_body(kk, carry):
        for b in range(3):
            stripe(kk * 3 + b, b)
        return carry

    lax.fori_loop(0, n_stripes // 3, trip_body, 0)
    stripe(n_stripes - 2, (n_stripes - 2) % 3)
    stripe(n_stripes - 1, (n_stripes - 1) % 3)

    for ob in range(2):
        pltpu.make_async_copy(
            out_bufs[ob], out_hbm.at[pl.ds(base, SR), pl.ds(ob * QW, QW)],
            osems[ob]).wait()


def kernel(u, inv_perm):
    B, D = u.shape
    mesh = plsc.VectorSubcoreMesh(
        core_axis_name="c", subcore_axis_name="s",
        num_cores=NC, num_subcores=NS,
    )
    f = pl.kernel(
        lambda *refs: _permute_body(B, D, *refs),
        out_type=jax.ShapeDtypeStruct((B, D), jnp.float32),
        mesh=mesh,
        compiler_params=pltpu.CompilerParams(
            use_tc_tiling_on_sc=True, needs_layout_passes=False,
        ),
        scratch_types=[
            pltpu.VMEM((D,), jnp.int32),
            pltpu.VMEM((SR, D), jnp.float32),
            pltpu.VMEM((SR, D), jnp.float32),
            pltpu.VMEM((SR, D), jnp.float32),
            pltpu.VMEM((SR, QW), jnp.float32),
            pltpu.VMEM((SR, QW), jnp.float32),
            pltpu.SemaphoreType.DMA,
            pltpu.SemaphoreType.DMA,
            pltpu.SemaphoreType.DMA,
            pltpu.SemaphoreType.DMA,
            pltpu.SemaphoreType.DMA,
        ],
    )
    return f(u, inv_perm.astype(jnp.int32))


# speedup vs baseline: 1.0538x; 1.0538x over previous
import jax
import jax.numpy as jnp
from jax import lax
from jax.experimental import pallas as pl
from jax.experimental.pallas import tpu as pltpu
from jax.experimental.pallas import tpu_sc as plsc

NC = 2
NS = 16
NW = NC * NS
LANES = 16
SR = 8
HALF = 2048


def _permute_body(B, D, u_hbm, perm_hbm, out_hbm,
                  idx_v, in0, in1, ob0, ob1,
                  isem0, isem1, osem0, osem1):
    rows_per_w = B // NW
    n_stripes = rows_per_w // SR

    wid = lax.axis_index("s") * NC + lax.axis_index("c")
    base = wid * rows_per_w

    in_bufs, isems = (in0, in1), (isem0, isem1)
    out_bufs, osems = (ob0, ob1), (osem0, osem1)

    pltpu.sync_copy(perm_hbm, idx_v)

    for b in range(2):
        pltpu.async_copy(u_hbm.at[pl.ds(base + b * SR, SR)],
                         in_bufs[b], isems[b])

    def stripe(k, b):
        row0 = base + k * SR
        in_v = in_bufs[b]
        pltpu.make_async_copy(u_hbm.at[pl.ds(row0, SR)],
                              in_v, isems[b]).wait()
        for h in range(2):
            m = 2 * k + h
            ob = h
            out_v = out_bufs[ob]

            @pl.when(m >= 2)
            def _():
                pltpu.make_async_copy(
                    out_v, out_hbm.at[pl.ds(row0, SR), pl.ds(h * HALF, HALF)],
                    osems[ob]).wait()

            @plsc.parallel_loop(0, HALF // LANES, unroll=2)
            def _(j):
                j16 = j * LANES
                idx16 = idx_v[pl.ds(h * HALF + j16, LANES)]
                for r in range(SR):
                    rvec = jnp.full((LANES,), r, jnp.int32)
                    vals = plsc.load_gather(in_v, [rvec, idx16])
                    out_v[r, pl.ds(j16, LANES)] = vals

            pltpu.async_copy(
                out_v, out_hbm.at[pl.ds(row0, SR), pl.ds(h * HALF, HALF)],
                osems[ob])

        @pl.when(k + 2 < n_stripes)
        def _():
            pltpu.async_copy(u_hbm.at[pl.ds(row0 + 2 * SR, SR)],
                             in_v, isems[b])

    def pair_body(kk, carry):
        for b in range(2):
            stripe(kk * 2 + b, b)
        return carry

    lax.fori_loop(0, n_stripes // 2, pair_body, 0)

    for ob in range(2):
        pltpu.make_async_copy(
            out_bufs[ob], out_hbm.at[pl.ds(base, SR), pl.ds(ob * HALF, HALF)],
            osems[ob]).wait()


def kernel(u, inv_perm):
    B, D = u.shape
    mesh = plsc.VectorSubcoreMesh(
        core_axis_name="c", subcore_axis_name="s",
        num_cores=NC, num_subcores=NS,
    )
    f = pl.kernel(
        lambda *refs: _permute_body(B, D, *refs),
        out_type=jax.ShapeDtypeStruct((B, D), jnp.float32),
        mesh=mesh,
        compiler_params=pltpu.CompilerParams(
            use_tc_tiling_on_sc=True, needs_layout_passes=False,
        ),
        scratch_types=[
            pltpu.VMEM((D,), jnp.int32),
            pltpu.VMEM((SR, D), jnp.float32),
            pltpu.VMEM((SR, D), jnp.float32),
            pltpu.VMEM((SR, HALF), jnp.float32),
            pltpu.VMEM((SR, HALF), jnp.float32),
            pltpu.SemaphoreType.DMA,
            pltpu.SemaphoreType.DMA,
            pltpu.SemaphoreType.DMA,
            pltpu.SemaphoreType.DMA,
        ],
    )
    return f(u, inv_perm.astype(jnp.int32))
